# baseline (speedup 1.0000x reference)
import functools

import jax
import jax.numpy as jnp
from jax import lax
from jax.experimental import pallas as pl
from jax.experimental.pallas import tpu as pltpu
from jax.experimental.pallas import tpu_sc as plsc

EMBED_DIM = 64


def kernel(heads, relations, tails, entity_embeddings, relation_embeddings):
    B = heads.shape[0]
    D = entity_embeddings.shape[1]
    assert D == EMBED_DIM

    info = plsc.get_sparse_core_info()
    NC, NS, L = info.num_cores, info.num_subcores, info.num_lanes
    NW = NC * NS
    assert B % (8 * NW) == 0
    bpw = B // NW

    mesh = plsc.VectorSubcoreMesh(core_axis_name="c", subcore_axis_name="s")

    @functools.partial(
        pl.kernel,
        mesh=mesh,
        out_type=jax.ShapeDtypeStruct((B,), jnp.float32),
        compiler_params=pltpu.CompilerParams(
            needs_layout_passes=False, use_tc_tiling_on_sc=False),
        scratch_types=[
            pltpu.VMEM((bpw,), jnp.int32),
            pltpu.VMEM((bpw,), jnp.int32),
            pltpu.VMEM((bpw,), jnp.int32),
            pltpu.VMEM((bpw, D), jnp.float32),
            pltpu.VMEM((bpw, D), jnp.float32),
            pltpu.VMEM((bpw, D), jnp.float32),
            pltpu.VMEM((bpw,), jnp.float32),
            pltpu.SemaphoreType.DMA,
        ],
    )
    def trans_e(heads_hbm, rel_hbm, tails_hbm, ent_hbm, relemb_hbm, out_hbm,
                hid_v, rid_v, tid_v, h_v, r_v, t_v, o_v, sem):
        wid = lax.axis_index("s") * NC + lax.axis_index("c")
        base = wid * bpw

        pltpu.sync_copy(heads_hbm.at[pl.ds(base, bpw)], hid_v)
        pltpu.sync_copy(rel_hbm.at[pl.ds(base, bpw)], rid_v)
        pltpu.sync_copy(tails_hbm.at[pl.ds(base, bpw)], tid_v)

        ch = pltpu.async_copy(ent_hbm.at[hid_v], h_v, sem)
        cr = pltpu.async_copy(relemb_hbm.at[rid_v], r_v, sem)
        ct = pltpu.async_copy(ent_hbm.at[tid_v], t_v, sem)
        ch.wait()
        cr.wait()
        ct.wait()

        lane = lax.iota(jnp.int32, L)
        last_lane = lane == (L - 1)

        def row(i, carry):
            acc = jnp.zeros((L,), jnp.float32)
            for q in range(D // L):
                sl = pl.ds(q * L, L)
                dv = h_v[i, sl] + r_v[i, sl] - t_v[i, sl]
                acc = acc + dv * dv
            c = plsc.cumsum(acc)
            plsc.store_scatter(o_v, [jnp.full((L,), i, jnp.int32)], c,
                               mask=last_lane)
            return carry

        lax.fori_loop(0, bpw, row, 0)

        def newton_sqrt(i, carry):
            sl = pl.ds(i * L, L)
            x = o_v[sl]
            xi = plsc.bitcast(x, jnp.int32)
            y = plsc.bitcast((xi >> 1) + jnp.int32(0x1FBD1DF5), jnp.float32)
            y = 0.5 * (y + x / y)
            y = 0.5 * (y + x / y)
            y = 0.5 * (y + x / y)
            o_v[sl] = y
            return carry

        lax.fori_loop(0, bpw // L, newton_sqrt, 0)

        pltpu.sync_copy(o_v, out_hbm.at[pl.ds(base, bpw)])

    return trans_e(heads, relations, tails, entity_embeddings, relation_embeddings)
